# baseline (speedup 1.0000x reference)
import functools

import jax
import jax.numpy as jnp
from jax import lax
from jax.experimental import pallas as pl
from jax.experimental.pallas import tpu as pltpu
from jax.experimental.pallas import tpu_sc as plsc

DIM = 64
LANES = 128
HPAD = 56
SUPER = 256
N_WORKERS = 32


@functools.partial(jax.jit, static_argnums=(2,))
def _gather_rows(idx, table, n_super):
    assert n_super % 2 == 0 and n_super >= 4
    per_w = n_super * SUPER

    @functools.partial(
        pl.kernel,
        out_type=jax.ShapeDtypeStruct((N_WORKERS * per_w, LANES), jnp.float32),
        mesh=plsc.VectorSubcoreMesh(core_axis_name="c", subcore_axis_name="s"),
        scratch_types=[
            pltpu.VMEM((n_super, SUPER), jnp.int32),
            pltpu.VMEM((2, SUPER, LANES), jnp.float32),
            pltpu.SemaphoreType.DMA,
            pltpu.SemaphoreType.DMA,
            pltpu.SemaphoreType.DMA,
            pltpu.SemaphoreType.DMA,
        ],
        compiler_params=pltpu.CompilerParams(use_tc_tiling_on_sc=False),
    )
    def k(idx_hbm, table_hbm, out_hbm, idx_v, rows, gsem0, gsem1, osem0, osem1):
        wid = lax.axis_index("s") * 2 + lax.axis_index("c")
        pltpu.sync_copy(idx_hbm.at[wid], idx_v)
        base = wid * per_w
        gsems = (gsem0, gsem1)
        osems = (osem0, osem1)

        def idx_slice(g):
            return idx_v.at[g]

        def out_slice(g):
            return out_hbm.at[pl.ds(base + g * SUPER, SUPER)]

        def fire_gather(g, grp):
            pltpu.async_copy(table_hbm.at[idx_slice(g)], rows.at[grp], gsems[grp])

        def drain_gather(g, grp):
            pltpu.make_async_copy(
                table_hbm.at[idx_slice(g)], rows.at[grp], gsems[grp]
            ).wait()

        def fire_out(g, grp):
            pltpu.async_copy(rows.at[grp], out_slice(g), osems[grp])

        def drain_out(g, grp):
            pltpu.make_async_copy(rows.at[grp], out_slice(g), osems[grp]).wait()

        def step(g, cur):
            nxt = 1 - cur
            drain_gather(g, cur)
            fire_out(g, cur)
            drain_out(g - 1, nxt)
            fire_gather(g + 1, nxt)

        fire_gather(0, 0)
        drain_gather(0, 0)
        fire_out(0, 0)
        fire_gather(1, 1)

        @pl.loop(0, (n_super - 2) // 2)
        def _(p):
            g = 1 + 2 * p
            step(g, 1)
            step(g + 1, 0)

        g_last = n_super - 1
        drain_gather(g_last, 1)
        fire_out(g_last, 1)
        drain_out(g_last - 1, 0)
        drain_out(g_last, 1)

    return k(idx, table)


def kernel(token_ids, weight):
    B, H = token_ids.shape
    V = weight.shape[0]
    table = jnp.pad(weight, ((0, 0), (0, LANES - DIM)))
    idx = jnp.pad(token_ids.astype(jnp.int32), ((0, 0), (0, HPAD - H)))
    total = B * HPAD
    per = N_WORKERS * SUPER * 2
    assert total % per == 0
    n_super = total // (N_WORKERS * SUPER)
    idx = idx.reshape(N_WORKERS, n_super, SUPER)
    out = _gather_rows(idx, table, n_super)
    return out.reshape(B, HPAD, LANES)[:, :H, :DIM]


# speedup vs baseline: 5.1287x; 5.1287x over previous
import functools

import jax
import jax.numpy as jnp
from jax import lax
from jax.experimental import pallas as pl
from jax.experimental.pallas import tpu as pltpu
from jax.experimental.pallas import tpu_sc as plsc

DIM = 64
LANES = 128
HPAD = 56
SUPER = 256
N_WORKERS = 32


@functools.partial(jax.jit, static_argnums=(2,))
def _gather_rows(idx, table, n_super):
    assert n_super % 2 == 0 and n_super >= 4
    per_w = n_super * SUPER

    @functools.partial(
        pl.kernel,
        out_type=jax.ShapeDtypeStruct((N_WORKERS * per_w, LANES), jnp.float32),
        mesh=plsc.VectorSubcoreMesh(core_axis_name="c", subcore_axis_name="s"),
        scratch_types=[
            pltpu.VMEM((n_super, SUPER), jnp.int32),
            pltpu.VMEM((2, SUPER, LANES), jnp.float32),
            pltpu.SemaphoreType.DMA,
            pltpu.SemaphoreType.DMA,
            pltpu.SemaphoreType.DMA,
            pltpu.SemaphoreType.DMA,
        ],
        compiler_params=pltpu.CompilerParams(use_tc_tiling_on_sc=False),
    )
    def k(idx_hbm, table_hbm, out_hbm, idx_v, rows, gsem0, gsem1, osem0, osem1):
        wid = lax.axis_index("s") * 2 + lax.axis_index("c")
        pltpu.sync_copy(idx_hbm.at[wid], idx_v)
        base = wid * per_w
        gsems = (gsem0, gsem1)
        osems = (osem0, osem1)

        def idx_slice(g):
            return idx_v.at[g]

        def out_slice(g):
            return out_hbm.at[pl.ds(base + g * SUPER, SUPER)]

        def fire_gather(g, grp):
            pltpu.async_copy(table_hbm.at[idx_slice(g)], rows.at[grp], gsems[grp])

        def drain_gather(g, grp):
            pltpu.make_async_copy(
                table_hbm.at[idx_slice(g)], rows.at[grp], gsems[grp]
            ).wait()

        def fire_out(g, grp):
            pltpu.async_copy(rows.at[grp], out_slice(g), osems[grp])

        def drain_out(g, grp):
            pltpu.make_async_copy(rows.at[grp], out_slice(g), osems[grp]).wait()

        def step(g, cur):
            nxt = 1 - cur
            drain_gather(g, cur)
            fire_out(g, cur)
            drain_out(g - 1, nxt)
            fire_gather(g + 1, nxt)

        fire_gather(0, 0)
        drain_gather(0, 0)
        fire_out(0, 0)
        fire_gather(1, 1)

        @pl.loop(0, (n_super - 2) // 2)
        def _(p):
            g = 1 + 2 * p
            step(g, 1)
            step(g + 1, 0)

        g_last = n_super - 1
        drain_gather(g_last, 1)
        fire_out(g_last, 1)
        drain_out(g_last - 1, 0)
        drain_out(g_last, 1)

    return k(idx, table)


def kernel(token_ids, weight):
    B, H = token_ids.shape
    V = weight.shape[0]
    table = jnp.pad(weight, ((0, 0), (0, LANES - DIM)))
    filler = jnp.arange(B * (HPAD - H), dtype=jnp.int32).reshape(B, HPAD - H) % V
    idx = jnp.concatenate([token_ids.astype(jnp.int32), filler], axis=1)
    total = B * HPAD
    per = N_WORKERS * SUPER * 2
    assert total % per == 0
    n_super = total // (N_WORKERS * SUPER)
    idx = idx.reshape(N_WORKERS, n_super, SUPER)
    out = _gather_rows(idx, table, n_super)
    return out.reshape(B, HPAD, LANES)[:, :H, :DIM]


# speedup vs baseline: 6.1056x; 1.1905x over previous
import functools

import jax
import jax.numpy as jnp
from jax import lax
from jax.experimental import pallas as pl
from jax.experimental.pallas import tpu as pltpu
from jax.experimental.pallas import tpu_sc as plsc

DIM = 64
LANES = 128
HPAD = 56
SUPER = 512
N_WORKERS = 32


@functools.partial(jax.jit, static_argnums=(2,))
def _gather_rows(idx, table, n_super):
    assert n_super % 2 == 0 and n_super >= 4
    per_w = n_super * SUPER

    @functools.partial(
        pl.kernel,
        out_type=jax.ShapeDtypeStruct((N_WORKERS * per_w, LANES), jnp.float32),
        mesh=plsc.VectorSubcoreMesh(core_axis_name="c", subcore_axis_name="s"),
        scratch_types=[
            pltpu.VMEM((n_super, SUPER), jnp.int32),
            pltpu.VMEM((2, SUPER, DIM), jnp.float32),
            pltpu.SemaphoreType.DMA,
            pltpu.SemaphoreType.DMA,
            pltpu.SemaphoreType.DMA,
            pltpu.SemaphoreType.DMA,
        ],
        compiler_params=pltpu.CompilerParams(use_tc_tiling_on_sc=False),
    )
    def k(idx_hbm, table_hbm, out_hbm, idx_v, rows, gsem0, gsem1, osem0, osem1):
        wid = lax.axis_index("s") * 2 + lax.axis_index("c")
        pltpu.sync_copy(idx_hbm.at[wid], idx_v)
        base = wid * per_w
        gsems = (gsem0, gsem1)
        osems = (osem0, osem1)

        def out_slice(g):
            return out_hbm.at[pl.ds(base + g * SUPER, SUPER), pl.ds(0, DIM)]

        def fire_gather(g, grp):
            pltpu.async_copy(table_hbm.at[idx_v.at[g]], rows.at[grp], gsems[grp])

        def drain_gather(g, grp):
            pltpu.make_async_copy(
                table_hbm.at[idx_v.at[g]], rows.at[grp], gsems[grp]
            ).wait()

        def fire_out(g, grp):
            pltpu.async_copy(rows.at[grp], out_slice(g), osems[grp])

        def drain_out(g, grp):
            pltpu.make_async_copy(rows.at[grp], out_slice(g), osems[grp]).wait()

        def step(g, cur):
            nxt = 1 - cur
            drain_gather(g, cur)
            fire_out(g, cur)
            drain_out(g - 1, nxt)
            fire_gather(g + 1, nxt)

        fire_gather(0, 0)
        drain_gather(0, 0)
        fire_out(0, 0)
        fire_gather(1, 1)

        @pl.loop(0, (n_super - 2) // 2)
        def _(p):
            g = 1 + 2 * p
            step(g, 1)
            step(g + 1, 0)

        g_last = n_super - 1
        drain_gather(g_last, 1)
        fire_out(g_last, 1)
        drain_out(g_last - 1, 0)
        drain_out(g_last, 1)

    return k(idx, table)


def kernel(token_ids, weight):
    B, H = token_ids.shape
    V = weight.shape[0]
    table = jnp.pad(weight, ((0, 0), (0, LANES - DIM))).reshape(2 * V, DIM)
    filler = jnp.arange(B * (HPAD - H), dtype=jnp.int32).reshape(B, HPAD - H) % V
    idx = jnp.concatenate([token_ids.astype(jnp.int32), filler], axis=1) * 2
    total = B * HPAD
    per = N_WORKERS * SUPER * 2
    assert total % per == 0
    n_super = total // (N_WORKERS * SUPER)
    idx = idx.reshape(N_WORKERS, n_super, SUPER)
    out = _gather_rows(idx, table, n_super)
    return out.reshape(B, HPAD, LANES)[:, :H, :DIM]
